# initial kernel scaffold (unmeasured)
import jax
import jax.numpy as jnp
from jax import lax
from jax.experimental import pallas as pl
from jax.experimental.pallas import tpu as pltpu

N_DEV = 32
E_PER = 2
N_EXP = N_DEV * E_PER
T = 256
D = 128
H = 256


def kernel(x, router_W, route_idx, expert_W, shared_W):
    def body(x_ref, router_ref, route_ref, expert_ref, shared_ref, out_ref,
             gathered_ref, send_sems, recv_sems):
        my = lax.axis_index("i")
        left = (my - 1) % N_DEV
        right = (my + 1) % N_DEV

        barrier_sem = pltpu.get_barrier_semaphore()
        for nbr in (left, right):
            pl.semaphore_signal(
                barrier_sem, inc=1,
                device_id=(nbr,), device_id_type=pl.DeviceIdType.MESH,
            )
        pl.semaphore_wait(barrier_sem, 2)

        xv = x_ref[:, :]
        scores = jnp.dot(xv, router_ref[:, :], preferred_element_type=jnp.float32)
        m = jnp.max(scores, axis=-1, keepdims=True)
        p = jnp.exp(scores - m)
        probs = p / jnp.sum(p, axis=-1, keepdims=True)
        route = route_ref[:, :]
        out_ref[:, :] = jnp.dot(xv, shared_ref[:, :],
                                preferred_element_type=jnp.float32)

        gathered_ref[my] = expert_ref[:, :, :]

        def accumulate(origin):
            for k in range(E_PER):
                e = E_PER * origin + k
                w = gathered_ref[origin, k]
                p_col = lax.dynamic_slice_in_dim(probs, e, 1, axis=1)
                coef = jnp.where(route == e, p_col, 0.0)
                out_ref[:, :] += coef * jnp.dot(
                    xv, w, preferred_element_type=jnp.float32)

        def copy_for(origin):
            return pltpu.make_async_remote_copy(
                src_ref=gathered_ref.at[origin],
                dst_ref=gathered_ref.at[origin],
                send_sem=send_sems.at[origin],
                recv_sem=recv_sems.at[origin],
                device_id=(right,),
                device_id_type=pl.DeviceIdType.MESH,
            )

        for h in range(N_DEV - 1):
            s = (my - h) % N_DEV
            r = (my - h - 1) % N_DEV
            send = copy_for(s)
            send.start()
            accumulate(s)
            send.wait_send()
            copy_for(r).wait_recv()

        accumulate((my + 1) % N_DEV)

    return pl.pallas_call(
        body,
        out_shape=jax.ShapeDtypeStruct((T, H), jnp.float32),
        in_specs=[
            pl.BlockSpec(memory_space=pltpu.VMEM),
            pl.BlockSpec(memory_space=pltpu.VMEM),
            pl.BlockSpec(memory_space=pltpu.VMEM),
            pl.BlockSpec(memory_space=pltpu.VMEM),
            pl.BlockSpec(memory_space=pltpu.VMEM),
        ],
        out_specs=pl.BlockSpec(memory_space=pltpu.VMEM),
        scratch_shapes=[
            pltpu.VMEM((N_DEV, E_PER, D, H), jnp.float32),
            pltpu.SemaphoreType.DMA((N_DEV,)),
            pltpu.SemaphoreType.DMA((N_DEV,)),
        ],
        compiler_params=pltpu.CompilerParams(collective_id=0),
    )(x, router_W, route_idx, expert_W, shared_W)


# baseline (device time: 153368 ns/iter reference)
import jax
import jax.numpy as jnp
from jax import lax
from jax.experimental import pallas as pl
from jax.experimental.pallas import tpu as pltpu

N_DEV = 32
E_PER = 2
N_EXP = N_DEV * E_PER
T = 256
D = 128
H = 256


def kernel(x, router_W, route_idx, expert_W, shared_W):
    def body(x_ref, router_ref, route_ref, expert_ref, shared_ref, out_ref,
             gathered_ref, send_sems, recv_sems):
        my = lax.axis_index("i")
        left = (my - 1) % N_DEV
        right = (my + 1) % N_DEV

        barrier_sem = pltpu.get_barrier_semaphore()
        for nbr in (left, right):
            pl.semaphore_signal(
                barrier_sem, inc=1,
                device_id=(nbr,), device_id_type=pl.DeviceIdType.MESH,
            )
        pl.semaphore_wait(barrier_sem, 2)

        xv = x_ref[:, :]
        scores = jnp.dot(xv, router_ref[:, :], preferred_element_type=jnp.float32)
        m = jnp.max(scores, axis=-1, keepdims=True)
        p = jnp.exp(scores - m)
        probs = p / jnp.sum(p, axis=-1, keepdims=True)
        route = route_ref[:, :]
        col_ids = lax.broadcasted_iota(jnp.int32, (T, N_EXP), 1)
        out_ref[:, :] = jnp.dot(xv, shared_ref[:, :],
                                preferred_element_type=jnp.float32)

        gathered_ref[my] = expert_ref[:, :, :]

        def accumulate(origin):
            for k in range(E_PER):
                e = E_PER * origin + k
                w = gathered_ref[origin, k]
                p_col = jnp.sum(jnp.where(col_ids == e, probs, 0.0),
                                axis=1, keepdims=True)
                coef = jnp.where(route == e, p_col, 0.0)
                out_ref[:, :] += coef * jnp.dot(
                    xv, w, preferred_element_type=jnp.float32)

        def copy_for(origin):
            return pltpu.make_async_remote_copy(
                src_ref=gathered_ref.at[origin],
                dst_ref=gathered_ref.at[origin],
                send_sem=send_sems.at[origin],
                recv_sem=recv_sems.at[origin],
                device_id=(right,),
                device_id_type=pl.DeviceIdType.MESH,
            )

        for h in range(N_DEV - 1):
            s = (my - h) % N_DEV
            r = (my - h - 1) % N_DEV
            send = copy_for(s)
            send.start()
            accumulate(s)
            send.wait_send()
            copy_for(r).wait_recv()

        accumulate((my + 1) % N_DEV)

    return pl.pallas_call(
        body,
        out_shape=jax.ShapeDtypeStruct((T, H), jnp.float32),
        in_specs=[
            pl.BlockSpec(memory_space=pltpu.VMEM),
            pl.BlockSpec(memory_space=pltpu.VMEM),
            pl.BlockSpec(memory_space=pltpu.VMEM),
            pl.BlockSpec(memory_space=pltpu.VMEM),
            pl.BlockSpec(memory_space=pltpu.VMEM),
        ],
        out_specs=pl.BlockSpec(memory_space=pltpu.VMEM),
        scratch_shapes=[
            pltpu.VMEM((N_DEV, E_PER, D, H), jnp.float32),
            pltpu.SemaphoreType.DMA((N_DEV,)),
            pltpu.SemaphoreType.DMA((N_DEV,)),
        ],
        compiler_params=pltpu.CompilerParams(collective_id=0),
    )(x, router_W, route_idx, expert_W, shared_W)


# device time: 122755 ns/iter; 1.2494x vs baseline; 1.2494x over previous
import jax
import jax.numpy as jnp
from jax import lax
from jax.experimental import pallas as pl
from jax.experimental.pallas import tpu as pltpu

N_DEV = 32
E_PER = 2
N_EXP = N_DEV * E_PER
T = 256
D = 128
H = 256

FWD_HOPS = N_DEV // 2
BWD_HOPS = N_DEV - 1 - FWD_HOPS


def kernel(x, router_W, route_idx, expert_W, shared_W):
    def body(x_ref, router_ref, route_ref, expert_ref, shared_ref, out_ref,
             gathered_ref, send_fwd, recv_fwd, send_bwd, recv_bwd):
        my = lax.axis_index("i")
        left = (my - 1) % N_DEV
        right = (my + 1) % N_DEV

        barrier_sem = pltpu.get_barrier_semaphore()
        for nbr in (left, right):
            pl.semaphore_signal(
                barrier_sem, inc=1,
                device_id=(nbr,), device_id_type=pl.DeviceIdType.MESH,
            )
        pl.semaphore_wait(barrier_sem, 2)

        gathered_ref[my] = expert_ref[:, :, :]

        def copy(origin, dest, ssems, rsems):
            return pltpu.make_async_remote_copy(
                src_ref=gathered_ref.at[origin],
                dst_ref=gathered_ref.at[origin],
                send_sem=ssems.at[origin],
                recv_sem=rsems.at[origin],
                device_id=(dest,),
                device_id_type=pl.DeviceIdType.MESH,
            )

        copy(my, right, send_fwd, recv_fwd).start()
        copy(my, left, send_bwd, recv_bwd).start()

        xv = x_ref[:, :]
        scores = jnp.dot(xv, router_ref[:, :], preferred_element_type=jnp.float32)
        m = jnp.max(scores, axis=-1, keepdims=True)
        p = jnp.exp(scores - m)
        probs = p / jnp.sum(p, axis=-1, keepdims=True)
        route = route_ref[:, :]
        col_ids = lax.broadcasted_iota(jnp.int32, (T, N_EXP), 1)
        out_ref[:, :] = jnp.dot(xv, shared_ref[:, :],
                                preferred_element_type=jnp.float32)

        def accumulate(origin):
            for k in range(E_PER):
                e = E_PER * origin + k
                w = gathered_ref[origin, k]
                p_col = jnp.sum(jnp.where(col_ids == e, probs, 0.0),
                                axis=1, keepdims=True)
                coef = jnp.where(route == e, p_col, 0.0)
                out_ref[:, :] += coef * jnp.dot(
                    xv, w, preferred_element_type=jnp.float32)

        accumulate(my)

        for h in range(FWD_HOPS):
            f = (my - h - 1) % N_DEV
            b = (my + h + 1) % N_DEV
            copy(f, right, send_fwd, recv_fwd).wait_recv()
            if h < BWD_HOPS:
                copy(b, left, send_bwd, recv_bwd).wait_recv()
            if h < FWD_HOPS - 1:
                copy(f, right, send_fwd, recv_fwd).start()
            if h < BWD_HOPS - 1:
                copy(b, left, send_bwd, recv_bwd).start()
            accumulate(f)
            if h < BWD_HOPS:
                accumulate(b)

        for h in range(FWD_HOPS):
            copy((my - h) % N_DEV, right, send_fwd, recv_fwd).wait_send()
        for h in range(BWD_HOPS):
            copy((my + h) % N_DEV, left, send_bwd, recv_bwd).wait_send()

    return pl.pallas_call(
        body,
        out_shape=jax.ShapeDtypeStruct((T, H), jnp.float32),
        in_specs=[
            pl.BlockSpec(memory_space=pltpu.VMEM),
            pl.BlockSpec(memory_space=pltpu.VMEM),
            pl.BlockSpec(memory_space=pltpu.VMEM),
            pl.BlockSpec(memory_space=pltpu.VMEM),
            pl.BlockSpec(memory_space=pltpu.VMEM),
        ],
        out_specs=pl.BlockSpec(memory_space=pltpu.VMEM),
        scratch_shapes=[
            pltpu.VMEM((N_DEV, E_PER, D, H), jnp.float32),
            pltpu.SemaphoreType.DMA((N_DEV,)),
            pltpu.SemaphoreType.DMA((N_DEV,)),
            pltpu.SemaphoreType.DMA((N_DEV,)),
            pltpu.SemaphoreType.DMA((N_DEV,)),
        ],
        compiler_params=pltpu.CompilerParams(collective_id=0),
    )(x, router_W, route_idx, expert_W, shared_W)


# device time: 84496 ns/iter; 1.8151x vs baseline; 1.4528x over previous
import jax
import jax.numpy as jnp
from jax import lax
from jax.experimental import pallas as pl
from jax.experimental.pallas import tpu as pltpu

N_DEV = 32
E_PER = 2
N_EXP = N_DEV * E_PER
T = 256
D = 128
H = 256

FWD_HOPS = N_DEV // 2
BWD_HOPS = N_DEV - 1 - FWD_HOPS


def _ring_tables():
    logical = []
    for z in range(4):
        for y in range(4):
            row = [(0, y, z), (1, y, z)]
            if y % 2:
                row.reverse()
            logical.extend(row)
    plane = [(0, 0), (1, 0), (2, 0), (3, 0),
             (3, 1), (2, 1), (1, 1), (0, 1),
             (0, 2), (1, 2), (2, 2), (3, 2),
             (3, 3), (2, 3), (1, 3), (0, 3)]
    cycle = [(0, y, z) for (y, z) in plane] + \
            [(1, y, z) for (y, z) in reversed(plane)]
    log_index = {c: i for i, c in enumerate(logical)}
    pos2log = [log_index[c] for c in cycle]
    log2pos = [0] * N_DEV
    for p, l in enumerate(pos2log):
        log2pos[l] = p
    return log2pos, pos2log


_LOG2POS, _POS2LOG = _ring_tables()


def kernel(x, router_W, route_idx, expert_W, shared_W):
    def body(x_ref, router_ref, route_ref, expert_ref, shared_ref,
             log2pos_ref, pos2log_ref, out_ref,
             gathered_ref, send_fwd, recv_fwd, send_bwd, recv_bwd):
        my = lax.axis_index("i")

        idx32 = lax.broadcasted_iota(jnp.int32, (1, N_DEV), 1)
        log2pos_t = log2pos_ref[:, :]
        pos2log_t = pos2log_ref[:, :]

        def lut(table, i):
            return jnp.sum(jnp.where(idx32 == i, table, 0))

        pos = lut(log2pos_t, my)
        right = lut(pos2log_t, (pos + 1) % N_DEV)
        left = lut(pos2log_t, (pos - 1) % N_DEV)

        barrier_sem = pltpu.get_barrier_semaphore()
        for nbr in (left, right):
            pl.semaphore_signal(
                barrier_sem, inc=1,
                device_id=(nbr,), device_id_type=pl.DeviceIdType.MESH,
            )
        pl.semaphore_wait(barrier_sem, 2)

        gathered_ref[my] = expert_ref[:, :, :]

        def copy(origin, dest, ssems, rsems):
            return pltpu.make_async_remote_copy(
                src_ref=gathered_ref.at[origin],
                dst_ref=gathered_ref.at[origin],
                send_sem=ssems.at[origin],
                recv_sem=rsems.at[origin],
                device_id=(dest,),
                device_id_type=pl.DeviceIdType.MESH,
            )

        copy(my, right, send_fwd, recv_fwd).start()
        copy(my, left, send_bwd, recv_bwd).start()

        xv = x_ref[:, :]
        scores = jnp.dot(xv, router_ref[:, :], preferred_element_type=jnp.float32)
        m = jnp.max(scores, axis=-1, keepdims=True)
        p = jnp.exp(scores - m)
        probs = p / jnp.sum(p, axis=-1, keepdims=True)
        route = route_ref[:, :]
        col_ids = lax.broadcasted_iota(jnp.int32, (T, N_EXP), 1)
        out_ref[:, :] = jnp.dot(xv, shared_ref[:, :],
                                preferred_element_type=jnp.float32)

        def accumulate(origin):
            wpair = gathered_ref[origin].reshape(E_PER * D, H)
            parts = []
            for k in range(E_PER):
                e = E_PER * origin + k
                p_col = jnp.sum(jnp.where(col_ids == e, probs, 0.0),
                                axis=1, keepdims=True)
                coef = jnp.where(route == e, p_col, 0.0)
                parts.append(coef * xv)
            xc = jnp.concatenate(parts, axis=1)
            out_ref[:, :] += jnp.dot(xc, wpair,
                                     preferred_element_type=jnp.float32)

        accumulate(my)

        for h in range(FWD_HOPS):
            f = lut(pos2log_t, (pos - h - 1) % N_DEV)
            b = lut(pos2log_t, (pos + h + 1) % N_DEV)
            copy(f, right, send_fwd, recv_fwd).wait_recv()
            if h < BWD_HOPS:
                copy(b, left, send_bwd, recv_bwd).wait_recv()
            if h < FWD_HOPS - 1:
                copy(f, right, send_fwd, recv_fwd).start()
            if h < BWD_HOPS - 1:
                copy(b, left, send_bwd, recv_bwd).start()
            accumulate(f)
            if h < BWD_HOPS:
                accumulate(b)

        for h in range(FWD_HOPS):
            copy(lut(pos2log_t, (pos - h) % N_DEV),
                 right, send_fwd, recv_fwd).wait_send()
        for h in range(BWD_HOPS):
            copy(lut(pos2log_t, (pos + h) % N_DEV),
                 left, send_bwd, recv_bwd).wait_send()

    return pl.pallas_call(
        body,
        out_shape=jax.ShapeDtypeStruct((T, H), jnp.float32),
        in_specs=[
            pl.BlockSpec(memory_space=pltpu.VMEM),
            pl.BlockSpec(memory_space=pltpu.VMEM),
            pl.BlockSpec(memory_space=pltpu.VMEM),
            pl.BlockSpec(memory_space=pltpu.VMEM),
            pl.BlockSpec(memory_space=pltpu.VMEM),
            pl.BlockSpec(memory_space=pltpu.VMEM),
            pl.BlockSpec(memory_space=pltpu.VMEM),
        ],
        out_specs=pl.BlockSpec(memory_space=pltpu.VMEM),
        scratch_shapes=[
            pltpu.VMEM((N_DEV, E_PER, D, H), jnp.float32),
            pltpu.SemaphoreType.DMA((N_DEV,)),
            pltpu.SemaphoreType.DMA((N_DEV,)),
            pltpu.SemaphoreType.DMA((N_DEV,)),
            pltpu.SemaphoreType.DMA((N_DEV,)),
        ],
        compiler_params=pltpu.CompilerParams(collective_id=0),
    )(x, router_W, route_idx, expert_W, shared_W,
      jnp.asarray(_LOG2POS, dtype=jnp.int32).reshape(1, N_DEV),
      jnp.asarray(_POS2LOG, dtype=jnp.int32).reshape(1, N_DEV))


# device time: 63114 ns/iter; 2.4300x vs baseline; 1.3388x over previous
import jax
import jax.numpy as jnp
from jax import lax
from jax.experimental import pallas as pl
from jax.experimental.pallas import tpu as pltpu

N_DEV = 32
E_PER = 2
N_EXP = N_DEV * E_PER
T = 256
D = 128
H = 256

FWD_HOPS = N_DEV // 2
BWD_HOPS = N_DEV - 1 - FWD_HOPS


def _ring_tables():
    logical = []
    for z in range(4):
        for y in range(4):
            row = [(0, y, z), (1, y, z)]
            if y % 2:
                row.reverse()
            logical.extend(row)
    plane = [(0, 0), (1, 0), (2, 0), (3, 0),
             (3, 1), (2, 1), (1, 1), (0, 1),
             (0, 2), (1, 2), (2, 2), (3, 2),
             (3, 3), (2, 3), (1, 3), (0, 3)]
    cycle = [(0, y, z) for (y, z) in plane] + \
            [(1, y, z) for (y, z) in reversed(plane)]
    log_index = {c: i for i, c in enumerate(logical)}
    pos2log = [log_index[c] for c in cycle]
    log2pos = [0] * N_DEV
    for p, l in enumerate(pos2log):
        log2pos[l] = p
    return log2pos, pos2log


_LOG2POS, _POS2LOG = _ring_tables()


def kernel(x, router_W, route_idx, expert_W, shared_W):
    def body(x_ref, router_ref, route_ref, expert_ref, shared_ref,
             log2pos_ref, pos2log_ref, out_ref,
             gathered_ref, send_fwd, recv_fwd, send_bwd, recv_bwd):
        my = lax.axis_index("i")

        idx32 = lax.broadcasted_iota(jnp.int32, (1, N_DEV), 1)
        log2pos_t = log2pos_ref[:, :]
        pos2log_t = pos2log_ref[:, :]

        def lut(table, i):
            return jnp.sum(jnp.where(idx32 == i, table, 0))

        pos = lut(log2pos_t, my)
        right = lut(pos2log_t, (pos + 1) % N_DEV)
        left = lut(pos2log_t, (pos - 1) % N_DEV)

        barrier_sem = pltpu.get_barrier_semaphore()
        for nbr in (left, right):
            pl.semaphore_signal(
                barrier_sem, inc=1,
                device_id=(nbr,), device_id_type=pl.DeviceIdType.MESH,
            )
        pl.semaphore_wait(barrier_sem, 2)

        gathered_ref[my] = expert_ref[:, :, :]

        def copy(origin, k, dest, ssems, rsems):
            return pltpu.make_async_remote_copy(
                src_ref=gathered_ref.at[origin, k],
                dst_ref=gathered_ref.at[origin, k],
                send_sem=ssems.at[origin, k],
                recv_sem=rsems.at[origin, k],
                device_id=(dest,),
                device_id_type=pl.DeviceIdType.MESH,
            )

        for k in range(E_PER):
            copy(my, k, right, send_fwd, recv_fwd).start()
            copy(my, k, left, send_bwd, recv_bwd).start()

        xv = x_ref[:, :]
        scores = jnp.dot(xv, router_ref[:, :], preferred_element_type=jnp.float32)
        m = jnp.max(scores, axis=-1, keepdims=True)
        p = jnp.exp(scores - m)
        probs = p / jnp.sum(p, axis=-1, keepdims=True)
        route = route_ref[:, :]
        col_ids = lax.broadcasted_iota(jnp.int32, (T, N_EXP), 1)
        out_ref[:, :] = jnp.dot(xv, shared_ref[:, :],
                                preferred_element_type=jnp.float32)

        def accumulate(origin):
            wpair = gathered_ref[origin].reshape(E_PER * D, H)
            parts = []
            for k in range(E_PER):
                e = E_PER * origin + k
                p_col = jnp.sum(jnp.where(col_ids == e, probs, 0.0),
                                axis=1, keepdims=True)
                coef = jnp.where(route == e, p_col, 0.0)
                parts.append(coef * xv)
            xc = jnp.concatenate(parts, axis=1)
            out_ref[:, :] += jnp.dot(xc, wpair,
                                     preferred_element_type=jnp.float32)

        accumulate(my)

        for h in range(FWD_HOPS):
            f = lut(pos2log_t, (pos - h - 1) % N_DEV)
            b = lut(pos2log_t, (pos + h + 1) % N_DEV)
            for k in range(E_PER):
                copy(f, k, right, send_fwd, recv_fwd).wait_recv()
                if h < FWD_HOPS - 1:
                    copy(f, k, right, send_fwd, recv_fwd).start()
                if h < BWD_HOPS:
                    copy(b, k, left, send_bwd, recv_bwd).wait_recv()
                    if h < BWD_HOPS - 1:
                        copy(b, k, left, send_bwd, recv_bwd).start()
            accumulate(f)
            if h < BWD_HOPS:
                accumulate(b)

        for h in range(FWD_HOPS):
            for k in range(E_PER):
                copy(lut(pos2log_t, (pos - h) % N_DEV),
                     k, right, send_fwd, recv_fwd).wait_send()
        for h in range(BWD_HOPS):
            for k in range(E_PER):
                copy(lut(pos2log_t, (pos + h) % N_DEV),
                     k, left, send_bwd, recv_bwd).wait_send()

    return pl.pallas_call(
        body,
        out_shape=jax.ShapeDtypeStruct((T, H), jnp.float32),
        in_specs=[
            pl.BlockSpec(memory_space=pltpu.VMEM),
            pl.BlockSpec(memory_space=pltpu.VMEM),
            pl.BlockSpec(memory_space=pltpu.VMEM),
            pl.BlockSpec(memory_space=pltpu.VMEM),
            pl.BlockSpec(memory_space=pltpu.VMEM),
            pl.BlockSpec(memory_space=pltpu.VMEM),
            pl.BlockSpec(memory_space=pltpu.VMEM),
        ],
        out_specs=pl.BlockSpec(memory_space=pltpu.VMEM),
        scratch_shapes=[
            pltpu.VMEM((N_DEV, E_PER, D, H), jnp.float32),
            pltpu.SemaphoreType.DMA((N_DEV, E_PER)),
            pltpu.SemaphoreType.DMA((N_DEV, E_PER)),
            pltpu.SemaphoreType.DMA((N_DEV, E_PER)),
            pltpu.SemaphoreType.DMA((N_DEV, E_PER)),
        ],
        compiler_params=pltpu.CompilerParams(collective_id=0),
    )(x, router_W, route_idx, expert_W, shared_W,
      jnp.asarray(_LOG2POS, dtype=jnp.int32).reshape(1, N_DEV),
      jnp.asarray(_POS2LOG, dtype=jnp.int32).reshape(1, N_DEV))


# device time: 59514 ns/iter; 2.5770x vs baseline; 1.0605x over previous
import jax
import jax.numpy as jnp
from jax import lax
from jax.experimental import pallas as pl
from jax.experimental.pallas import tpu as pltpu

N_DEV = 32
E_PER = 2
N_EXP = N_DEV * E_PER
T = 256
D = 128
H = 256

FWD_HOPS = N_DEV // 2
BWD_HOPS = N_DEV - 1 - FWD_HOPS


def _ring_tables():
    logical = []
    for z in range(4):
        for y in range(4):
            row = [(0, y, z), (1, y, z)]
            if y % 2:
                row.reverse()
            logical.extend(row)
    plane = [(0, 0), (1, 0), (2, 0), (3, 0),
             (3, 1), (2, 1), (1, 1), (0, 1),
             (0, 2), (1, 2), (2, 2), (3, 2),
             (3, 3), (2, 3), (1, 3), (0, 3)]
    cycle = [(0, y, z) for (y, z) in plane] + \
            [(1, y, z) for (y, z) in reversed(plane)]
    log_index = {c: i for i, c in enumerate(logical)}
    pos2log = [log_index[c] for c in cycle]
    log2pos = [0] * N_DEV
    for p, l in enumerate(pos2log):
        log2pos[l] = p
    return log2pos, pos2log


_LOG2POS, _POS2LOG = _ring_tables()


def kernel(x, router_W, route_idx, expert_W, shared_W):
    def body(x_ref, router_ref, route_ref, expert_ref, shared_ref,
             log2pos_ref, pos2log_ref, out_ref,
             gathered_ref, send_fwd, recv_fwd, send_bwd, recv_bwd):
        my = lax.axis_index("i")

        idx32 = lax.broadcasted_iota(jnp.int32, (1, N_DEV), 1)
        log2pos_t = log2pos_ref[:, :]
        pos2log_t = pos2log_ref[:, :]

        def lut(table, i):
            return jnp.sum(jnp.where(idx32 == i, table, 0))

        pos = lut(log2pos_t, my)
        right = lut(pos2log_t, (pos + 1) % N_DEV)
        left = lut(pos2log_t, (pos - 1) % N_DEV)

        barrier_sem = pltpu.get_barrier_semaphore()
        for nbr in (left, right):
            pl.semaphore_signal(
                barrier_sem, inc=1,
                device_id=(nbr,), device_id_type=pl.DeviceIdType.MESH,
            )
        pl.semaphore_wait(barrier_sem, 2)

        gathered_ref[my] = expert_ref[:, :, :]

        def copy(origin, k, j, dest, ssems, rsems):
            return pltpu.make_async_remote_copy(
                src_ref=gathered_ref.at[origin, k, pl.ds(j * (D // 2), D // 2)],
                dst_ref=gathered_ref.at[origin, k, pl.ds(j * (D // 2), D // 2)],
                send_sem=ssems.at[origin, k, j],
                recv_sem=rsems.at[origin, k, j],
                device_id=(dest,),
                device_id_type=pl.DeviceIdType.MESH,
            )

        for k in range(E_PER):
            for j in range(2):
                copy(my, k, j, right, send_fwd, recv_fwd).start()
                copy(my, k, j, left, send_bwd, recv_bwd).start()

        xv = x_ref[:, :]
        scores = jnp.dot(xv, router_ref[:, :], preferred_element_type=jnp.float32)
        m = jnp.max(scores, axis=-1, keepdims=True)
        p = jnp.exp(scores - m)
        probs = p / jnp.sum(p, axis=-1, keepdims=True)
        route = route_ref[:, :]
        col_ids = lax.broadcasted_iota(jnp.int32, (T, N_EXP), 1)
        out_ref[:, :] = jnp.dot(xv, shared_ref[:, :],
                                preferred_element_type=jnp.float32)

        def accumulate(origin):
            wpair = gathered_ref[origin].reshape(E_PER * D, H)
            parts = []
            for k in range(E_PER):
                e = E_PER * origin + k
                p_col = jnp.sum(jnp.where(col_ids == e, probs, 0.0),
                                axis=1, keepdims=True)
                coef = jnp.where(route == e, p_col, 0.0)
                parts.append(coef * xv)
            xc = jnp.concatenate(parts, axis=1)
            out_ref[:, :] += jnp.dot(xc, wpair,
                                     preferred_element_type=jnp.float32)

        accumulate(my)

        for h in range(FWD_HOPS):
            f = lut(pos2log_t, (pos - h - 1) % N_DEV)
            b = lut(pos2log_t, (pos + h + 1) % N_DEV)
            for k in range(E_PER):
                for j in range(2):
                    copy(f, k, j, right, send_fwd, recv_fwd).wait_recv()
                    if h < FWD_HOPS - 1:
                        copy(f, k, j, right, send_fwd, recv_fwd).start()
                    if h < BWD_HOPS:
                        copy(b, k, j, left, send_bwd, recv_bwd).wait_recv()
                        if h < BWD_HOPS - 1:
                            copy(b, k, j, left, send_bwd, recv_bwd).start()
            accumulate(f)
            if h < BWD_HOPS:
                accumulate(b)

        for h in range(FWD_HOPS):
            for k in range(E_PER):
                for j in range(2):
                    copy(lut(pos2log_t, (pos - h) % N_DEV),
                         k, j, right, send_fwd, recv_fwd).wait_send()
        for h in range(BWD_HOPS):
            for k in range(E_PER):
                for j in range(2):
                    copy(lut(pos2log_t, (pos + h) % N_DEV),
                         k, j, left, send_bwd, recv_bwd).wait_send()

    return pl.pallas_call(
        body,
        out_shape=jax.ShapeDtypeStruct((T, H), jnp.float32),
        in_specs=[
            pl.BlockSpec(memory_space=pltpu.VMEM),
            pl.BlockSpec(memory_space=pltpu.VMEM),
            pl.BlockSpec(memory_space=pltpu.VMEM),
            pl.BlockSpec(memory_space=pltpu.VMEM),
            pl.BlockSpec(memory_space=pltpu.VMEM),
            pl.BlockSpec(memory_space=pltpu.VMEM),
            pl.BlockSpec(memory_space=pltpu.VMEM),
        ],
        out_specs=pl.BlockSpec(memory_space=pltpu.VMEM),
        scratch_shapes=[
            pltpu.VMEM((N_DEV, E_PER, D, H), jnp.float32),
            pltpu.SemaphoreType.DMA((N_DEV, E_PER, 2)),
            pltpu.SemaphoreType.DMA((N_DEV, E_PER, 2)),
            pltpu.SemaphoreType.DMA((N_DEV, E_PER, 2)),
            pltpu.SemaphoreType.DMA((N_DEV, E_PER, 2)),
        ],
        compiler_params=pltpu.CompilerParams(collective_id=0),
    )(x, router_W, route_idx, expert_W, shared_W,
      jnp.asarray(_LOG2POS, dtype=jnp.int32).reshape(1, N_DEV),
      jnp.asarray(_POS2LOG, dtype=jnp.int32).reshape(1, N_DEV))
